# baseline (speedup 1.0000x reference)
import functools

import jax
import jax.numpy as jnp
import numpy as np
from jax import lax
from jax.experimental import pallas as pl
from jax.experimental.pallas import tpu as pltpu
from jax.experimental.pallas import tpu_sc as plsc


def _threefry2x32_np(k1, k2, x0, x1):
    rots = ((13, 15, 26, 6), (17, 29, 16, 24))
    ks = (
        np.uint32(k1),
        np.uint32(k2),
        np.uint32(k1) ^ np.uint32(k2) ^ np.uint32(0x1BD11BDA),
    )
    x0 = (x0 + ks[0]).astype(np.uint32)
    x1 = (x1 + ks[1]).astype(np.uint32)
    for i in range(5):
        for r in rots[i % 2]:
            x0 = (x0 + x1).astype(np.uint32)
            x1 = ((x1 << np.uint32(r)) | (x1 >> np.uint32(32 - r))) ^ x0
        x0 = (x0 + ks[(i + 1) % 3]).astype(np.uint32)
        x1 = (x1 + ks[(i + 2) % 3] + np.uint32(i + 1)).astype(np.uint32)
    return x0, x1


def _randint_key0_np(B, N):
    s1, s2 = _threefry2x32_np(0, 0, np.zeros(2, np.uint32), np.arange(2, dtype=np.uint32))
    zeros, iota = np.zeros(B, np.uint32), np.arange(B, dtype=np.uint32)
    h1, h2 = _threefry2x32_np(s1[0], s2[0], zeros, iota)
    l1, l2 = _threefry2x32_np(s1[1], s2[1], zeros, iota)
    higher, lower = h1 ^ h2, l1 ^ l2
    span = np.uint32(N)
    mult = np.uint32(np.uint32(2**16) % span)
    mult = np.uint32(mult * mult) % span
    return ((higher % span) * mult + lower % span) % span


def _flat_idx_np(B, N):
    return _randint_key0_np(B, N).astype(np.int64) + np.arange(B, dtype=np.int64) * N


def _make_gather(D: int, B: int, flat_idx):
    mesh = plsc.ScalarSubcoreMesh(axis_name="c", num_cores=1)

    @functools.partial(
        pl.kernel,
        mesh=mesh,
        out_type=jax.ShapeDtypeStruct((B, D), jnp.float32),
        scratch_types=[pltpu.SemaphoreType.DMA],
    )
    def gather(table_hbm, out_hbm, sem):
        for b in range(B):
            pltpu.async_copy(
                table_hbm.at[pl.ds(int(flat_idx[b]), 1)],
                out_hbm.at[pl.ds(b, 1)],
                sem,
            )
        pltpu.make_async_copy(
            table_hbm.at[pl.ds(0, B)],
            out_hbm.at[pl.ds(0, B)],
            sem,
        ).wait()

    return gather


def kernel(x):
    B, N, D = x.shape
    table = x.reshape(B * N, D)
    return _make_gather(D, B, _flat_idx_np(B, N))(table)


# speedup vs baseline: 1.0985x; 1.0985x over previous
import functools

import jax
import jax.numpy as jnp
import numpy as np
from jax import lax
from jax.experimental import pallas as pl
from jax.experimental.pallas import tpu as pltpu
from jax.experimental.pallas import tpu_sc as plsc


def _threefry2x32_np(k1, k2, x0, x1):
    rots = ((13, 15, 26, 6), (17, 29, 16, 24))
    ks = (
        np.uint32(k1),
        np.uint32(k2),
        np.uint32(k1) ^ np.uint32(k2) ^ np.uint32(0x1BD11BDA),
    )
    x0 = (x0 + ks[0]).astype(np.uint32)
    x1 = (x1 + ks[1]).astype(np.uint32)
    for i in range(5):
        for r in rots[i % 2]:
            x0 = (x0 + x1).astype(np.uint32)
            x1 = ((x1 << np.uint32(r)) | (x1 >> np.uint32(32 - r))) ^ x0
        x0 = (x0 + ks[(i + 1) % 3]).astype(np.uint32)
        x1 = (x1 + ks[(i + 2) % 3] + np.uint32(i + 1)).astype(np.uint32)
    return x0, x1


def _randint_key0_np(B, N):
    s1, s2 = _threefry2x32_np(0, 0, np.zeros(2, np.uint32), np.arange(2, dtype=np.uint32))
    zeros, iota = np.zeros(B, np.uint32), np.arange(B, dtype=np.uint32)
    h1, h2 = _threefry2x32_np(s1[0], s2[0], zeros, iota)
    l1, l2 = _threefry2x32_np(s1[1], s2[1], zeros, iota)
    higher, lower = h1 ^ h2, l1 ^ l2
    span = np.uint32(N)
    mult = np.uint32(np.uint32(2**16) % span)
    mult = np.uint32(mult * mult) % span
    return ((higher % span) * mult + lower % span) % span


def _flat_idx_np(B, N):
    return _randint_key0_np(B, N).astype(np.int64) + np.arange(B, dtype=np.int64) * N


def _make_gather(D: int, B: int):
    n_workers = 16
    b_per_w = B // n_workers
    mesh = plsc.VectorSubcoreMesh(
        core_axis_name="c", subcore_axis_name="s", num_cores=1
    )

    @functools.partial(
        pl.kernel,
        mesh=mesh,
        out_type=jax.ShapeDtypeStruct((B, D), jnp.float32),
        scratch_types=[
            pltpu.VMEM((b_per_w,), jnp.int32),
            pltpu.VMEM((b_per_w, D), jnp.float32),
            pltpu.SemaphoreType.DMA,
        ],
    )
    def gather(table_hbm, idx_hbm, out_hbm, idx_v, rows_v, sem):
        base = lax.axis_index("s") * b_per_w
        pltpu.sync_copy(idx_hbm.at[pl.ds(base, b_per_w)], idx_v)
        pltpu.async_copy(table_hbm.at[idx_v], rows_v, sem).wait()
        pltpu.sync_copy(rows_v, out_hbm.at[pl.ds(base, b_per_w)])

    return gather


def kernel(x):
    B, N, D = x.shape
    table = x.reshape(B * N, D)
    flat_idx = jnp.asarray(_flat_idx_np(B, N), dtype=jnp.int32)
    return _make_gather(D, B)(table, flat_idx)
